# baseline (speedup 1.0000x reference)
import functools

import jax
import jax.numpy as jnp
from jax.experimental import pallas as pl
from jax.experimental.pallas import tpu as pltpu

INPUT_DIM = 1024
OUTPUT_DIM = 1024
NUM_EXPERTS = 8
TOP_K = 2
BATCH = 2048

_PREC = jax.lax.Precision.HIGHEST


def _router_body(x_ref, wr_ref, br_ref, logits_ref, idx_ref, gates_ref):
    x = x_ref[...]
    wr = wr_ref[...]
    logits = jax.lax.dot_general(
        x, wr, (((1,), (1,)), ((), ())),
        preferred_element_type=jnp.float32)
    logits = logits + br_ref[...]
    logits_ref[...] = logits

    e_iota = jax.lax.broadcasted_iota(jnp.int32, logits.shape, 1)
    big = jnp.int32(NUM_EXPERTS)
    m1 = jnp.max(logits, axis=1, keepdims=True)
    i1 = jnp.min(jnp.where(logits == m1, e_iota, big), axis=1, keepdims=True)
    masked = jnp.where(e_iota == i1, -jnp.inf, logits)
    m2 = jnp.max(masked, axis=1, keepdims=True)
    i2 = jnp.min(jnp.where(masked == m2, e_iota, big), axis=1, keepdims=True)

    idx_ref[...] = jnp.concatenate([i1, i2], axis=1)
    gates_ref[...] = jnp.where(
        e_iota == i1, m1, jnp.where(e_iota == i2, m2, 0.0))


def _router(x, Wr, br):
    B = x.shape[0]
    E = Wr.shape[0]
    return pl.pallas_call(
        _router_body,
        out_shape=(
            jax.ShapeDtypeStruct((B, E), jnp.float32),
            jax.ShapeDtypeStruct((B, TOP_K), jnp.int32),
            jax.ShapeDtypeStruct((B, E), jnp.float32),
        ),
    )(x, Wr, br.reshape(1, E))


def _expert_body(x_ref, we_ref, be_ref, gates_ref, out_ref):
    x = x_ref[...]
    w = we_ref[0]
    acc = jax.lax.dot_general(
        x, w, (((1,), (1,)), ((), ())),
        preferred_element_type=jnp.float32)
    acc = jnp.maximum(acc + be_ref[0], 0.0)
    e = pl.program_id(0)
    gates = gates_ref[...]
    col = jax.lax.broadcasted_iota(jnp.int32, gates.shape, 1)
    g = jnp.sum(jnp.where(col == e, gates, 0.0), axis=1, keepdims=True)
    out_ref[0] = acc * g


def _experts_dense(x, We, be, gates):
    B, I = x.shape
    E, O, _ = We.shape
    BM = 256
    MT = B // BM
    return pl.pallas_call(
        _expert_body,
        grid=(E, MT),
        in_specs=[
            pl.BlockSpec((BM, I), lambda e, m: (m, 0)),
            pl.BlockSpec((1, O, I), lambda e, m: (e, 0, 0)),
            pl.BlockSpec((1, 1, O), lambda e, m: (e, 0, 0)),
            pl.BlockSpec((BM, NUM_EXPERTS), lambda e, m: (m, 0)),
        ],
        out_specs=pl.BlockSpec((1, BM, O), lambda e, m: (e, m, 0)),
        out_shape=jax.ShapeDtypeStruct((E, B, O), jnp.float32),
    )(x, We, be.reshape(E, 1, O), gates)


def kernel(x, Wr, br, We, be):
    logits, topk_idx, gates = _router(x, Wr, br)
    weighted = _experts_dense(x, We, be, gates)
    return (weighted, logits, topk_idx)


# speedup vs baseline: 1.2820x; 1.2820x over previous
import functools

import jax
import jax.numpy as jnp
from jax.experimental import pallas as pl
from jax.experimental.pallas import tpu as pltpu

INPUT_DIM = 1024
OUTPUT_DIM = 1024
NUM_EXPERTS = 8
TOP_K = 2
BATCH = 2048

_PREC = jax.lax.Precision.HIGHEST


def _router_body(x_ref, wr_ref, br_ref, logits_ref, idx_ref, gates_ref):
    x = x_ref[...]
    wr = wr_ref[...]
    logits = jax.lax.dot_general(
        x, wr, (((1,), (1,)), ((), ())),
        preferred_element_type=jnp.float32)
    logits = logits + br_ref[...]
    logits_ref[...] = logits

    e_iota = jax.lax.broadcasted_iota(jnp.int32, logits.shape, 1)
    big = jnp.int32(NUM_EXPERTS)
    m1 = jnp.max(logits, axis=1, keepdims=True)
    i1 = jnp.min(jnp.where(logits == m1, e_iota, big), axis=1, keepdims=True)
    masked = jnp.where(e_iota == i1, -jnp.inf, logits)
    m2 = jnp.max(masked, axis=1, keepdims=True)
    i2 = jnp.min(jnp.where(masked == m2, e_iota, big), axis=1, keepdims=True)

    idx_ref[...] = jnp.concatenate([i1, i2], axis=1)
    gates_ref[...] = jnp.where(
        e_iota == i1, m1, jnp.where(e_iota == i2, m2, 0.0))


def _router(x, Wr, br):
    B = x.shape[0]
    E = Wr.shape[0]
    return pl.pallas_call(
        _router_body,
        out_shape=(
            jax.ShapeDtypeStruct((B, E), jnp.float32),
            jax.ShapeDtypeStruct((B, TOP_K), jnp.int32),
            jax.ShapeDtypeStruct((B, E), jnp.float32),
        ),
    )(x, Wr, br.reshape(1, E))


def _expert_body(x_ref, we_ref, be_ref, gates_ref, out_ref):
    e = pl.program_id(0)
    m = pl.program_id(1)
    BM = out_ref.shape[1]
    x = x_ref[pl.ds(m * BM, BM), :]
    w = we_ref[e]
    acc = jax.lax.dot_general(
        x, w, (((1,), (1,)), ((), ())),
        preferred_element_type=jnp.float32)
    acc = jnp.maximum(acc + be_ref[e][None, :], 0.0)
    gates = gates_ref[pl.ds(m * BM, BM), :]
    col = jax.lax.broadcasted_iota(jnp.int32, gates.shape, 1)
    g = jnp.sum(jnp.where(col == e, gates, 0.0), axis=1, keepdims=True)
    out_ref[0] = acc * g


def _experts_dense(x, We, be, gates):
    B, I = x.shape
    E, O, _ = We.shape
    BM = 256
    MT = B // BM
    return pl.pallas_call(
        _expert_body,
        grid=(E, MT),
        in_specs=[
            pl.BlockSpec((B, I), lambda e, m: (0, 0)),
            pl.BlockSpec((E, O, I), lambda e, m: (0, 0, 0)),
            pl.BlockSpec((E, O), lambda e, m: (0, 0)),
            pl.BlockSpec((B, NUM_EXPERTS), lambda e, m: (0, 0)),
        ],
        out_specs=pl.BlockSpec((1, BM, O), lambda e, m: (e, m, 0)),
        out_shape=jax.ShapeDtypeStruct((E, B, O), jnp.float32),
    )(x, We, be, gates)


def kernel(x, Wr, br, We, be):
    logits, topk_idx, gates = _router(x, Wr, br)
    weighted = _experts_dense(x, We, be, gates)
    return (weighted, logits, topk_idx)
